# baseline (device time: 85424 ns/iter reference)
import jax
import jax.numpy as jnp
from jax import lax
from jax.experimental import pallas as pl
from jax.experimental.pallas import tpu as pltpu

P = 32
R = 8
C = 4
M = 1536
N = 1536
NH = N // 2
JR = M // R
SR = JR // C
HR = JR // 2

_MESH = pl.DeviceIdType.MESH
_BF16 = jnp.bfloat16
_F32 = jnp.float32


def _q_of(v):
    y = jnp.where(v == 0, 0, jnp.where(v <= 4, v - 1, 8 - v))
    x = jnp.where((v >= 1) & (v <= 4), 1, 0)
    return 2 * y + lax.rem(x + y, 2)


def _k_of(q):
    y = q // 2
    x = lax.rem(q + y, 2)
    return jnp.where(x == 1, y + 1, jnp.where(y == 0, 0, 8 - y))


def kernel(A, B):
    def body(a_ref, b_ref, out_ref, b_bf,
             p1p_land, p1p_stage, p1m_land, p1m_stage,
             p2ap_land, p2ap_stage, p2am_land, p2am_stage,
             p2bp_land, p2bp_stage, p2bm_land, p2bm_stage,
             p3p_land, p3p_stage, p3m_land, p3m_stage,
             p1p_ss, p1p_rs, p1m_ss, p1m_rs,
             p2ap_ss, p2ap_rs, p2am_ss, p2am_rs,
             p2bp_ss, p2bp_rs, p2bm_ss, p2bm_rs,
             p3p_ss, p3p_rs, p3m_ss, p3m_rs):
        my = lax.axis_index("i")
        r = my // R
        q = my % R
        k = _k_of(q)
        row_right = r * R + _q_of(lax.rem(k + 1, R))
        row_left = r * R + _q_of(lax.rem(k + R - 1, R))
        col_next = lax.rem(r + 1, C) * R + q
        col_prev = lax.rem(r + C - 1, C) * R + q

        lo = slice(0, NH)
        hi = slice(NH, N)

        barrier = pltpu.get_barrier_semaphore()
        for nbr in (row_left, row_right, col_prev, col_next):
            pl.semaphore_signal(
                barrier, inc=1, device_id=(nbr,), device_id_type=_MESH,
            )
        pl.semaphore_wait(barrier, 4)

        def copy(src, dst, ssem, rsem, dev):
            return pltpu.make_async_remote_copy(
                src_ref=src, dst_ref=dst, send_sem=ssem, recv_sem=rsem,
                device_id=(dev,), device_id_type=_MESH,
            )

        def p1_desc(s, u, plus):
            rows = pl.ds(u * HR, HR)
            if plus:
                return copy(p1p_stage.at[rows], p1p_land.at[s, rows],
                            p1p_ss.at[s, u], p1p_rs.at[s, u], row_right)
            return copy(p1m_stage.at[rows], p1m_land.at[s, rows],
                        p1m_ss.at[s, u], p1m_rs.at[s, u], row_left)

        def p2a_desc(s, plus):
            if plus:
                return copy(p2ap_stage, p2ap_land.at[s], p2ap_ss.at[s],
                            p2ap_rs.at[s], col_next)
            return copy(p2am_stage, p2am_land.at[s], p2am_ss.at[s],
                        p2am_rs.at[s], col_prev)

        def p2b_desc(t, plus):
            if plus:
                return copy(p2bp_stage if t == 0 else p2bp_land.at[t - 1],
                            p2bp_land.at[t], p2bp_ss.at[t], p2bp_rs.at[t],
                            col_next)
            return copy(p2bm_stage if t == 0 else p2bm_land.at[t - 1],
                        p2bm_land.at[t], p2bm_ss.at[t], p2bm_rs.at[t],
                        col_prev)

        def p3_desc(t, u, plus):
            rows = pl.ds(u * HR, HR)
            if plus:
                src = (p3p_stage.at[rows] if t == 0
                       else p3p_land.at[t - 1, rows])
                return copy(src, p3p_land.at[t, rows],
                            p3p_ss.at[t, u], p3p_rs.at[t, u], row_right)
            src = (p3m_stage.at[rows] if t == 0
                   else p3m_land.at[t - 1, rows])
            return copy(src, p3m_land.at[t, rows],
                        p3m_ss.at[t, u], p3m_rs.at[t, u], row_left)

        b_bf[...] = b_ref[...].astype(_BF16)
        ds_q = pl.ds(q * JR, JR)
        out_ref[ds_q, :] = jnp.dot(
            a_ref[ds_q, :].astype(_BF16), b_bf[...],
            preferred_element_type=_F32,
        )
        p1p_stage[...] = out_ref[ds_q, lo].astype(_BF16)
        p1m_stage[...] = out_ref[ds_q, hi].astype(_BF16)
        for u in (0, 1):
            p1_desc(0, u, True).start()
            p1_desc(0, u, False).start()
        for d in range(1, R):
            j = lax.rem(q + d, R)
            ds_j = pl.ds(j * JR, JR)
            out_ref[ds_j, :] = jnp.dot(
                a_ref[ds_j, :].astype(_BF16), b_bf[...],
                preferred_element_type=_F32,
            )

        for s in range(R - 1):
            for u in (0, 1):
                for plus in (True, False):
                    d = p1_desc(s, u, plus)
                    d.wait_recv()
                    if plus:
                        jj = _q_of(lax.rem(k - s - 1 + R, R))
                        half, land, stage = lo, p1p_land, p1p_stage
                    else:
                        jj = _q_of(lax.rem(k + s + 1, R))
                        half, land, stage = hi, p1m_land, p1m_stage
                    idx = pl.ds(jj * JR + u * HR, HR)
                    new = out_ref[idx, half] + (
                        land[s, u * HR:(u + 1) * HR].astype(_F32))
                    d.wait_send()
                    if s < R - 2:
                        stage[u * HR:(u + 1) * HR] = new.astype(_BF16)
                        p1_desc(s + 1, u, plus).start()
                    else:
                        out_ref[idx, half] = new

        jp_own = _q_of(lax.rem(k + 1, R))
        jm_own = _q_of(lax.rem(k + R - 1, R))
        base_p = jp_own * JR
        base_m = jm_own * JR

        p2ap_stage[...] = out_ref[pl.ds(base_p + r * SR, SR), lo].astype(_BF16)
        p2am_stage[...] = out_ref[pl.ds(base_m + r * SR, SR), hi].astype(_BF16)
        p2a_desc(0, True).start()
        p2a_desc(0, False).start()
        for s in range(C - 1):
            for plus in (True, False):
                d = p2a_desc(s, plus)
                d.wait_recv()
                if plus:
                    cc = lax.rem(r - s - 1 + C, C)
                    idx = pl.ds(base_p + cc * SR, SR)
                    half, land, stage = lo, p2ap_land, p2ap_stage
                else:
                    cc = lax.rem(r + s + 1, C)
                    idx = pl.ds(base_m + cc * SR, SR)
                    half, land, stage = hi, p2am_land, p2am_stage
                new = out_ref[idx, half] + land[s].astype(_F32)
                d.wait_send()
                if s < C - 2:
                    stage[...] = new.astype(_BF16)
                    p2a_desc(s + 1, plus).start()
                else:
                    out_ref[idx, half] = new

        cp_own = lax.rem(r + 1, C)
        cm_own = lax.rem(r + C - 1, C)

        p2bp_stage[...] = out_ref[
            pl.ds(base_p + cp_own * SR, SR), lo].astype(_BF16)
        p2bm_stage[...] = out_ref[
            pl.ds(base_m + cm_own * SR, SR), hi].astype(_BF16)
        p2b_desc(0, True).start()
        p2b_desc(0, False).start()
        for t in range(C - 1):
            for plus in (True, False):
                d = p2b_desc(t, plus)
                d.wait_recv()
                if t < C - 2:
                    p2b_desc(t + 1, plus).start()
                if plus:
                    cc = lax.rem(r - t + C, C)
                    out_ref[pl.ds(base_p + cc * SR, SR), lo] = (
                        p2bp_land[t].astype(_F32))
                else:
                    cc = lax.rem(r + t, C)
                    out_ref[pl.ds(base_m + cc * SR, SR), hi] = (
                        p2bm_land[t].astype(_F32))

        p3p_stage[...] = out_ref[pl.ds(base_p, JR), lo].astype(_BF16)
        p3m_stage[...] = out_ref[pl.ds(base_m, JR), hi].astype(_BF16)
        for u in (0, 1):
            p3_desc(0, u, True).start()
            p3_desc(0, u, False).start()
        for t in range(R - 1):
            for u in (0, 1):
                for plus in (True, False):
                    d = p3_desc(t, u, plus)
                    d.wait_recv()
                    if t < R - 2:
                        p3_desc(t + 1, u, plus).start()
                    us = slice(u * HR, (u + 1) * HR)
                    if plus:
                        jj = _q_of(lax.rem(k - t + R, R))
                        out_ref[pl.ds(jj * JR + u * HR, HR), lo] = jnp.maximum(
                            p3p_land[t, us].astype(_F32), 0.0)
                    else:
                        jj = _q_of(lax.rem(k + t, R))
                        out_ref[pl.ds(jj * JR + u * HR, HR), hi] = jnp.maximum(
                            p3m_land[t, us].astype(_F32), 0.0)

        out_ref[pl.ds(base_p, JR), lo] = jnp.maximum(
            out_ref[pl.ds(base_p, JR), lo], 0.0)
        out_ref[pl.ds(base_m, JR), hi] = jnp.maximum(
            out_ref[pl.ds(base_m, JR), hi], 0.0)

        for t in range(C - 1):
            p2b_desc(t, True).wait_send()
            p2b_desc(t, False).wait_send()
        for t in range(R - 1):
            for u in (0, 1):
                p3_desc(t, u, True).wait_send()
                p3_desc(t, u, False).wait_send()

    return pl.pallas_call(
        body,
        out_shape=jax.ShapeDtypeStruct((M, N), jnp.float32),
        in_specs=[
            pl.BlockSpec(memory_space=pltpu.VMEM),
            pl.BlockSpec(memory_space=pltpu.VMEM),
        ],
        out_specs=pl.BlockSpec(memory_space=pltpu.VMEM),
        scratch_shapes=[
            pltpu.VMEM((768, N), _BF16),
            pltpu.VMEM((R - 1, JR, NH), _BF16),
            pltpu.VMEM((JR, NH), _BF16),
            pltpu.VMEM((R - 1, JR, NH), _BF16),
            pltpu.VMEM((JR, NH), _BF16),
            pltpu.VMEM((C - 1, SR, NH), _BF16),
            pltpu.VMEM((SR, NH), _BF16),
            pltpu.VMEM((C - 1, SR, NH), _BF16),
            pltpu.VMEM((SR, NH), _BF16),
            pltpu.VMEM((C - 1, SR, NH), _BF16),
            pltpu.VMEM((SR, NH), _BF16),
            pltpu.VMEM((C - 1, SR, NH), _BF16),
            pltpu.VMEM((SR, NH), _BF16),
            pltpu.VMEM((R - 1, JR, NH), _BF16),
            pltpu.VMEM((JR, NH), _BF16),
            pltpu.VMEM((R - 1, JR, NH), _BF16),
            pltpu.VMEM((JR, NH), _BF16),
            pltpu.SemaphoreType.DMA((R - 1, 2)),
            pltpu.SemaphoreType.DMA((R - 1, 2)),
            pltpu.SemaphoreType.DMA((R - 1, 2)),
            pltpu.SemaphoreType.DMA((R - 1, 2)),
            pltpu.SemaphoreType.DMA((C - 1,)),
            pltpu.SemaphoreType.DMA((C - 1,)),
            pltpu.SemaphoreType.DMA((C - 1,)),
            pltpu.SemaphoreType.DMA((C - 1,)),
            pltpu.SemaphoreType.DMA((C - 1,)),
            pltpu.SemaphoreType.DMA((C - 1,)),
            pltpu.SemaphoreType.DMA((C - 1,)),
            pltpu.SemaphoreType.DMA((C - 1,)),
            pltpu.SemaphoreType.DMA((R - 1, 2)),
            pltpu.SemaphoreType.DMA((R - 1, 2)),
            pltpu.SemaphoreType.DMA((R - 1, 2)),
            pltpu.SemaphoreType.DMA((R - 1, 2)),
        ],
        compiler_params=pltpu.CompilerParams(collective_id=0),
    )(A, B)


# device time: 84398 ns/iter; 1.0122x vs baseline; 1.0122x over previous
import jax
import jax.numpy as jnp
from jax import lax
from jax.experimental import pallas as pl
from jax.experimental.pallas import tpu as pltpu

P = 32
R = 8
C = 4
M = 1536
N = 1536
NH = N // 2
JR = M // R
SR = JR // C
NU = 4
HR = JR // NU

_MESH = pl.DeviceIdType.MESH
_BF16 = jnp.bfloat16
_F32 = jnp.float32


def _q_of(v):
    y = jnp.where(v == 0, 0, jnp.where(v <= 4, v - 1, 8 - v))
    x = jnp.where((v >= 1) & (v <= 4), 1, 0)
    return 2 * y + lax.rem(x + y, 2)


def _k_of(q):
    y = q // 2
    x = lax.rem(q + y, 2)
    return jnp.where(x == 1, y + 1, jnp.where(y == 0, 0, 8 - y))


def kernel(A, B):
    def body(a_ref, b_ref, out_ref, b_bf,
             p1p_land, p1p_stage, p1m_land, p1m_stage,
             p2ap_land, p2ap_stage, p2am_land, p2am_stage,
             p2bp_land, p2bp_stage, p2bm_land, p2bm_stage,
             p3p_land, p3p_stage, p3m_land, p3m_stage,
             p1p_ss, p1p_rs, p1m_ss, p1m_rs,
             p2ap_ss, p2ap_rs, p2am_ss, p2am_rs,
             p2bp_ss, p2bp_rs, p2bm_ss, p2bm_rs,
             p3p_ss, p3p_rs, p3m_ss, p3m_rs):
        my = lax.axis_index("i")
        r = my // R
        q = my % R
        k = _k_of(q)
        row_right = r * R + _q_of(lax.rem(k + 1, R))
        row_left = r * R + _q_of(lax.rem(k + R - 1, R))
        col_next = lax.rem(r + 1, C) * R + q
        col_prev = lax.rem(r + C - 1, C) * R + q

        lo = slice(0, NH)
        hi = slice(NH, N)

        barrier = pltpu.get_barrier_semaphore()
        for nbr in (row_left, row_right, col_prev, col_next):
            pl.semaphore_signal(
                barrier, inc=1, device_id=(nbr,), device_id_type=_MESH,
            )
        pl.semaphore_wait(barrier, 4)

        def copy(src, dst, ssem, rsem, dev):
            return pltpu.make_async_remote_copy(
                src_ref=src, dst_ref=dst, send_sem=ssem, recv_sem=rsem,
                device_id=(dev,), device_id_type=_MESH,
            )

        def p1_desc(s, u, plus):
            rows = pl.ds(u * HR, HR)
            if plus:
                return copy(p1p_stage.at[rows], p1p_land.at[s, rows],
                            p1p_ss.at[s, u], p1p_rs.at[s, u], row_right)
            return copy(p1m_stage.at[rows], p1m_land.at[s, rows],
                        p1m_ss.at[s, u], p1m_rs.at[s, u], row_left)

        def p2a_desc(s, plus):
            if plus:
                return copy(p2ap_stage, p2ap_land.at[s], p2ap_ss.at[s],
                            p2ap_rs.at[s], col_next)
            return copy(p2am_stage, p2am_land.at[s], p2am_ss.at[s],
                        p2am_rs.at[s], col_prev)

        def p2b_desc(t, plus):
            if plus:
                return copy(p2bp_stage if t == 0 else p2bp_land.at[t - 1],
                            p2bp_land.at[t], p2bp_ss.at[t], p2bp_rs.at[t],
                            col_next)
            return copy(p2bm_stage if t == 0 else p2bm_land.at[t - 1],
                        p2bm_land.at[t], p2bm_ss.at[t], p2bm_rs.at[t],
                        col_prev)

        def p3_desc(t, u, plus):
            rows = pl.ds(u * HR, HR)
            if plus:
                src = (p3p_stage.at[rows] if t == 0
                       else p3p_land.at[t - 1, rows])
                return copy(src, p3p_land.at[t, rows],
                            p3p_ss.at[t, u], p3p_rs.at[t, u], row_right)
            src = (p3m_stage.at[rows] if t == 0
                   else p3m_land.at[t - 1, rows])
            return copy(src, p3m_land.at[t, rows],
                        p3m_ss.at[t, u], p3m_rs.at[t, u], row_left)

        b_bf[...] = b_ref[...].astype(_BF16)
        ds_q = pl.ds(q * JR, JR)
        out_ref[ds_q, :] = jnp.dot(
            a_ref[ds_q, :].astype(_BF16), b_bf[...],
            preferred_element_type=_F32,
        )
        p1p_stage[...] = out_ref[ds_q, lo].astype(_BF16)
        p1m_stage[...] = out_ref[ds_q, hi].astype(_BF16)
        for u in range(NU):
            p1_desc(0, u, True).start()
            p1_desc(0, u, False).start()
        for d in range(1, R):
            j = lax.rem(q + d, R)
            ds_j = pl.ds(j * JR, JR)
            out_ref[ds_j, :] = jnp.dot(
                a_ref[ds_j, :].astype(_BF16), b_bf[...],
                preferred_element_type=_F32,
            )

        for s in range(R - 1):
            for u in range(NU):
                for plus in (True, False):
                    d = p1_desc(s, u, plus)
                    d.wait_recv()
                    if plus:
                        jj = _q_of(lax.rem(k - s - 1 + R, R))
                        half, land, stage = lo, p1p_land, p1p_stage
                    else:
                        jj = _q_of(lax.rem(k + s + 1, R))
                        half, land, stage = hi, p1m_land, p1m_stage
                    idx = pl.ds(jj * JR + u * HR, HR)
                    new = out_ref[idx, half] + (
                        land[s, u * HR:(u + 1) * HR].astype(_F32))
                    d.wait_send()
                    if s < R - 2:
                        stage[u * HR:(u + 1) * HR] = new.astype(_BF16)
                        p1_desc(s + 1, u, plus).start()
                    else:
                        out_ref[idx, half] = new

        jp_own = _q_of(lax.rem(k + 1, R))
        jm_own = _q_of(lax.rem(k + R - 1, R))
        base_p = jp_own * JR
        base_m = jm_own * JR

        p2ap_stage[...] = out_ref[pl.ds(base_p + r * SR, SR), lo].astype(_BF16)
        p2am_stage[...] = out_ref[pl.ds(base_m + r * SR, SR), hi].astype(_BF16)
        p2a_desc(0, True).start()
        p2a_desc(0, False).start()
        for s in range(C - 1):
            for plus in (True, False):
                d = p2a_desc(s, plus)
                d.wait_recv()
                if plus:
                    cc = lax.rem(r - s - 1 + C, C)
                    idx = pl.ds(base_p + cc * SR, SR)
                    half, land, stage = lo, p2ap_land, p2ap_stage
                else:
                    cc = lax.rem(r + s + 1, C)
                    idx = pl.ds(base_m + cc * SR, SR)
                    half, land, stage = hi, p2am_land, p2am_stage
                new = out_ref[idx, half] + land[s].astype(_F32)
                d.wait_send()
                if s < C - 2:
                    stage[...] = new.astype(_BF16)
                    p2a_desc(s + 1, plus).start()
                else:
                    out_ref[idx, half] = new

        cp_own = lax.rem(r + 1, C)
        cm_own = lax.rem(r + C - 1, C)

        p2bp_stage[...] = out_ref[
            pl.ds(base_p + cp_own * SR, SR), lo].astype(_BF16)
        p2bm_stage[...] = out_ref[
            pl.ds(base_m + cm_own * SR, SR), hi].astype(_BF16)
        p2b_desc(0, True).start()
        p2b_desc(0, False).start()
        for t in range(C - 1):
            for plus in (True, False):
                d = p2b_desc(t, plus)
                d.wait_recv()
                if t < C - 2:
                    p2b_desc(t + 1, plus).start()
                if plus:
                    cc = lax.rem(r - t + C, C)
                    out_ref[pl.ds(base_p + cc * SR, SR), lo] = (
                        p2bp_land[t].astype(_F32))
                else:
                    cc = lax.rem(r + t, C)
                    out_ref[pl.ds(base_m + cc * SR, SR), hi] = (
                        p2bm_land[t].astype(_F32))

        p3p_stage[...] = out_ref[pl.ds(base_p, JR), lo].astype(_BF16)
        p3m_stage[...] = out_ref[pl.ds(base_m, JR), hi].astype(_BF16)
        for u in range(NU):
            p3_desc(0, u, True).start()
            p3_desc(0, u, False).start()
        for t in range(R - 1):
            for u in range(NU):
                for plus in (True, False):
                    d = p3_desc(t, u, plus)
                    d.wait_recv()
                    if t < R - 2:
                        p3_desc(t + 1, u, plus).start()
                    us = slice(u * HR, (u + 1) * HR)
                    if plus:
                        jj = _q_of(lax.rem(k - t + R, R))
                        out_ref[pl.ds(jj * JR + u * HR, HR), lo] = jnp.maximum(
                            p3p_land[t, us].astype(_F32), 0.0)
                    else:
                        jj = _q_of(lax.rem(k + t, R))
                        out_ref[pl.ds(jj * JR + u * HR, HR), hi] = jnp.maximum(
                            p3m_land[t, us].astype(_F32), 0.0)

        out_ref[pl.ds(base_p, JR), lo] = jnp.maximum(
            out_ref[pl.ds(base_p, JR), lo], 0.0)
        out_ref[pl.ds(base_m, JR), hi] = jnp.maximum(
            out_ref[pl.ds(base_m, JR), hi], 0.0)

        for t in range(C - 1):
            p2b_desc(t, True).wait_send()
            p2b_desc(t, False).wait_send()
        for t in range(R - 1):
            for u in range(NU):
                p3_desc(t, u, True).wait_send()
                p3_desc(t, u, False).wait_send()

    return pl.pallas_call(
        body,
        out_shape=jax.ShapeDtypeStruct((M, N), jnp.float32),
        in_specs=[
            pl.BlockSpec(memory_space=pltpu.VMEM),
            pl.BlockSpec(memory_space=pltpu.VMEM),
        ],
        out_specs=pl.BlockSpec(memory_space=pltpu.VMEM),
        scratch_shapes=[
            pltpu.VMEM((768, N), _BF16),
            pltpu.VMEM((R - 1, JR, NH), _BF16),
            pltpu.VMEM((JR, NH), _BF16),
            pltpu.VMEM((R - 1, JR, NH), _BF16),
            pltpu.VMEM((JR, NH), _BF16),
            pltpu.VMEM((C - 1, SR, NH), _BF16),
            pltpu.VMEM((SR, NH), _BF16),
            pltpu.VMEM((C - 1, SR, NH), _BF16),
            pltpu.VMEM((SR, NH), _BF16),
            pltpu.VMEM((C - 1, SR, NH), _BF16),
            pltpu.VMEM((SR, NH), _BF16),
            pltpu.VMEM((C - 1, SR, NH), _BF16),
            pltpu.VMEM((SR, NH), _BF16),
            pltpu.VMEM((R - 1, JR, NH), _BF16),
            pltpu.VMEM((JR, NH), _BF16),
            pltpu.VMEM((R - 1, JR, NH), _BF16),
            pltpu.VMEM((JR, NH), _BF16),
            pltpu.SemaphoreType.DMA((R - 1, NU)),
            pltpu.SemaphoreType.DMA((R - 1, NU)),
            pltpu.SemaphoreType.DMA((R - 1, NU)),
            pltpu.SemaphoreType.DMA((R - 1, NU)),
            pltpu.SemaphoreType.DMA((C - 1,)),
            pltpu.SemaphoreType.DMA((C - 1,)),
            pltpu.SemaphoreType.DMA((C - 1,)),
            pltpu.SemaphoreType.DMA((C - 1,)),
            pltpu.SemaphoreType.DMA((C - 1,)),
            pltpu.SemaphoreType.DMA((C - 1,)),
            pltpu.SemaphoreType.DMA((C - 1,)),
            pltpu.SemaphoreType.DMA((C - 1,)),
            pltpu.SemaphoreType.DMA((R - 1, NU)),
            pltpu.SemaphoreType.DMA((R - 1, NU)),
            pltpu.SemaphoreType.DMA((R - 1, NU)),
            pltpu.SemaphoreType.DMA((R - 1, NU)),
        ],
        compiler_params=pltpu.CompilerParams(collective_id=0),
    )(A, B)


# device time: 82098 ns/iter; 1.0405x vs baseline; 1.0280x over previous
import jax
import jax.numpy as jnp
from jax import lax
from jax.experimental import pallas as pl
from jax.experimental.pallas import tpu as pltpu

P = 32
R = 8
C = 4
M = 1536
N = 1536
NH = N // 2
JR = M // R
SR = JR // C
NU = 4
HR = JR // NU

_MESH = pl.DeviceIdType.MESH
_BF16 = jnp.bfloat16
_F32 = jnp.float32


def _q_of(v):
    y = jnp.where(v == 0, 0, jnp.where(v <= 4, v - 1, 8 - v))
    x = jnp.where((v >= 1) & (v <= 4), 1, 0)
    return 2 * y + lax.rem(x + y, 2)


def _k_of(q):
    y = q // 2
    x = lax.rem(q + y, 2)
    return jnp.where(x == 1, y + 1, jnp.where(y == 0, 0, 8 - y))


def kernel(A, B):
    def body(a_ref, b_ref, out_ref, b_bf,
             p1p_land, p1p_stage, p1m_land, p1m_stage,
             p2ap_land, p2ap_stage, p2am_land, p2am_stage,
             p2bp_land, p2bp_stage, p2bm_land, p2bm_stage,
             p3p_land, p3p_stage, p3m_land, p3m_stage,
             p1p_ss, p1p_rs, p1m_ss, p1m_rs,
             p2ap_ss, p2ap_rs, p2am_ss, p2am_rs,
             p2bp_ss, p2bp_rs, p2bm_ss, p2bm_rs,
             p3p_ss, p3p_rs, p3m_ss, p3m_rs):
        my = lax.axis_index("i")
        r = my // R
        q = my % R
        k = _k_of(q)
        row_right = r * R + _q_of(lax.rem(k + 1, R))
        row_left = r * R + _q_of(lax.rem(k + R - 1, R))
        col_next = lax.rem(r + 1, C) * R + q
        col_prev = lax.rem(r + C - 1, C) * R + q

        lo = slice(0, NH)
        hi = slice(NH, N)

        barrier = pltpu.get_barrier_semaphore()
        for nbr in (row_left, row_right, col_prev, col_next):
            pl.semaphore_signal(
                barrier, inc=1, device_id=(nbr,), device_id_type=_MESH,
            )
        pl.semaphore_wait(barrier, 4)

        def copy(src, dst, ssem, rsem, dev):
            return pltpu.make_async_remote_copy(
                src_ref=src, dst_ref=dst, send_sem=ssem, recv_sem=rsem,
                device_id=(dev,), device_id_type=_MESH,
            )

        def p1_desc(s, u, plus):
            rows = pl.ds(u * HR, HR)
            if plus:
                return copy(p1p_stage.at[rows], p1p_land.at[s, rows],
                            p1p_ss.at[s, u], p1p_rs.at[s, u], row_right)
            return copy(p1m_stage.at[rows], p1m_land.at[s, rows],
                        p1m_ss.at[s, u], p1m_rs.at[s, u], row_left)

        def p2a_desc(s, plus):
            if plus:
                return copy(p2ap_stage, p2ap_land.at[s], p2ap_ss.at[s],
                            p2ap_rs.at[s], col_next)
            return copy(p2am_stage, p2am_land.at[s], p2am_ss.at[s],
                        p2am_rs.at[s], col_prev)

        def p2b_desc(t, plus):
            if plus:
                return copy(p2bp_stage if t == 0 else p2bp_land.at[t - 1],
                            p2bp_land.at[t], p2bp_ss.at[t], p2bp_rs.at[t],
                            col_next)
            return copy(p2bm_stage if t == 0 else p2bm_land.at[t - 1],
                        p2bm_land.at[t], p2bm_ss.at[t], p2bm_rs.at[t],
                        col_prev)

        def p3_desc(t, u, plus):
            rows = pl.ds(u * HR, HR)
            if plus:
                src = (p3p_stage.at[rows] if t == 0
                       else p3p_land.at[t - 1, rows])
                return copy(src, p3p_land.at[t, rows],
                            p3p_ss.at[t, u], p3p_rs.at[t, u], row_right)
            src = (p3m_stage.at[rows] if t == 0
                   else p3m_land.at[t - 1, rows])
            return copy(src, p3m_land.at[t, rows],
                        p3m_ss.at[t, u], p3m_rs.at[t, u], row_left)

        def dot_chunk(j):
            ds_j = pl.ds(j * JR, JR)
            out_ref[ds_j, :] = jnp.dot(
                a_ref[ds_j, :].astype(_BF16), b_bf[...],
                preferred_element_type=_F32,
            )

        b_bf[...] = b_ref[...].astype(_BF16)
        dot_chunk(q)
        ds_q = pl.ds(q * JR, JR)
        p1p_stage[...] = out_ref[ds_q, lo].astype(_BF16)
        p1m_stage[...] = out_ref[ds_q, hi].astype(_BF16)
        for u in range(NU):
            p1_desc(0, u, True).start()
            p1_desc(0, u, False).start()
        dot_chunk(_q_of(lax.rem(k + 1, R)))
        dot_chunk(_q_of(lax.rem(k + R - 1, R)))

        for s in range(R - 1):
            if 0 < s <= 3:
                dot_chunk(_q_of(lax.rem(k + s + 1, R)))
                if s < 3:
                    dot_chunk(_q_of(lax.rem(k + R - s - 1, R)))
            for u in range(NU):
                for plus in (True, False):
                    d = p1_desc(s, u, plus)
                    d.wait_recv()
                    if plus:
                        jj = _q_of(lax.rem(k - s - 1 + R, R))
                        half, land, stage = lo, p1p_land, p1p_stage
                    else:
                        jj = _q_of(lax.rem(k + s + 1, R))
                        half, land, stage = hi, p1m_land, p1m_stage
                    idx = pl.ds(jj * JR + u * HR, HR)
                    new = out_ref[idx, half] + (
                        land[s, u * HR:(u + 1) * HR].astype(_F32))
                    d.wait_send()
                    if s < R - 2:
                        stage[u * HR:(u + 1) * HR] = new.astype(_BF16)
                        p1_desc(s + 1, u, plus).start()
                    else:
                        out_ref[idx, half] = new

        jp_own = _q_of(lax.rem(k + 1, R))
        jm_own = _q_of(lax.rem(k + R - 1, R))
        base_p = jp_own * JR
        base_m = jm_own * JR

        cp_own = lax.rem(r + 1, C)
        cm_own = lax.rem(r + C - 1, C)

        p2ap_stage[...] = out_ref[pl.ds(base_p + r * SR, SR), lo].astype(_BF16)
        p2am_stage[...] = out_ref[pl.ds(base_m + r * SR, SR), hi].astype(_BF16)
        p2a_desc(0, True).start()
        p2a_desc(0, False).start()
        for s in range(C - 1):
            for plus in (True, False):
                d = p2a_desc(s, plus)
                d.wait_recv()
                if plus:
                    cc = lax.rem(r - s - 1 + C, C)
                    idx = pl.ds(base_p + cc * SR, SR)
                    half, land, stage = lo, p2ap_land, p2ap_stage
                else:
                    cc = lax.rem(r + s + 1, C)
                    idx = pl.ds(base_m + cc * SR, SR)
                    half, land, stage = hi, p2am_land, p2am_stage
                new = out_ref[idx, half] + land[s].astype(_F32)
                d.wait_send()
                if s < C - 2:
                    stage[...] = new.astype(_BF16)
                    p2a_desc(s + 1, plus).start()
                else:
                    out_ref[idx, half] = new
                    nb = new.astype(_BF16)
                    if plus:
                        p2bp_stage[...] = nb
                        p3p_stage[pl.ds(cp_own * HR, HR)] = nb
                    else:
                        p2bm_stage[...] = nb
                        p3m_stage[pl.ds(cm_own * HR, HR)] = nb

        p2b_desc(0, True).start()
        p2b_desc(0, False).start()
        p3_desc(0, cp_own, True).start()
        p3_desc(0, cm_own, False).start()

        for t in range(C - 1):
            for plus in (True, False):
                d = p2b_desc(t, plus)
                d.wait_recv()
                if t < C - 2:
                    p2b_desc(t + 1, plus).start()
                if plus:
                    cc = lax.rem(r - t + C, C)
                    p3p_stage[pl.ds(cc * HR, HR)] = p2bp_land[t]
                    p3_desc(0, cc, True).start()
                    out_ref[pl.ds(base_p + cc * SR, SR), lo] = (
                        p2bp_land[t].astype(_F32))
                else:
                    cc = lax.rem(r + t, C)
                    p3m_stage[pl.ds(cc * HR, HR)] = p2bm_land[t]
                    p3_desc(0, cc, False).start()
                    out_ref[pl.ds(base_m + cc * SR, SR), hi] = (
                        p2bm_land[t].astype(_F32))

        for t in range(R - 1):
            for u in range(NU):
                for plus in (True, False):
                    d = p3_desc(t, u, plus)
                    d.wait_recv()
                    if t < R - 2:
                        p3_desc(t + 1, u, plus).start()
                    us = slice(u * HR, (u + 1) * HR)
                    if plus:
                        jj = _q_of(lax.rem(k - t + R, R))
                        out_ref[pl.ds(jj * JR + u * HR, HR), lo] = jnp.maximum(
                            p3p_land[t, us].astype(_F32), 0.0)
                    else:
                        jj = _q_of(lax.rem(k + t, R))
                        out_ref[pl.ds(jj * JR + u * HR, HR), hi] = jnp.maximum(
                            p3m_land[t, us].astype(_F32), 0.0)

        out_ref[pl.ds(base_p, JR), lo] = jnp.maximum(
            out_ref[pl.ds(base_p, JR), lo], 0.0)
        out_ref[pl.ds(base_m, JR), hi] = jnp.maximum(
            out_ref[pl.ds(base_m, JR), hi], 0.0)

        for t in range(C - 1):
            p2b_desc(t, True).wait_send()
            p2b_desc(t, False).wait_send()
        for t in range(R - 1):
            for u in range(NU):
                p3_desc(t, u, True).wait_send()
                p3_desc(t, u, False).wait_send()

    return pl.pallas_call(
        body,
        out_shape=jax.ShapeDtypeStruct((M, N), jnp.float32),
        in_specs=[
            pl.BlockSpec(memory_space=pltpu.VMEM),
            pl.BlockSpec(memory_space=pltpu.VMEM),
        ],
        out_specs=pl.BlockSpec(memory_space=pltpu.VMEM),
        scratch_shapes=[
            pltpu.VMEM((768, N), _BF16),
            pltpu.VMEM((R - 1, JR, NH), _BF16),
            pltpu.VMEM((JR, NH), _BF16),
            pltpu.VMEM((R - 1, JR, NH), _BF16),
            pltpu.VMEM((JR, NH), _BF16),
            pltpu.VMEM((C - 1, SR, NH), _BF16),
            pltpu.VMEM((SR, NH), _BF16),
            pltpu.VMEM((C - 1, SR, NH), _BF16),
            pltpu.VMEM((SR, NH), _BF16),
            pltpu.VMEM((C - 1, SR, NH), _BF16),
            pltpu.VMEM((SR, NH), _BF16),
            pltpu.VMEM((C - 1, SR, NH), _BF16),
            pltpu.VMEM((SR, NH), _BF16),
            pltpu.VMEM((R - 1, JR, NH), _BF16),
            pltpu.VMEM((JR, NH), _BF16),
            pltpu.VMEM((R - 1, JR, NH), _BF16),
            pltpu.VMEM((JR, NH), _BF16),
            pltpu.SemaphoreType.DMA((R - 1, NU)),
            pltpu.SemaphoreType.DMA((R - 1, NU)),
            pltpu.SemaphoreType.DMA((R - 1, NU)),
            pltpu.SemaphoreType.DMA((R - 1, NU)),
            pltpu.SemaphoreType.DMA((C - 1,)),
            pltpu.SemaphoreType.DMA((C - 1,)),
            pltpu.SemaphoreType.DMA((C - 1,)),
            pltpu.SemaphoreType.DMA((C - 1,)),
            pltpu.SemaphoreType.DMA((C - 1,)),
            pltpu.SemaphoreType.DMA((C - 1,)),
            pltpu.SemaphoreType.DMA((C - 1,)),
            pltpu.SemaphoreType.DMA((C - 1,)),
            pltpu.SemaphoreType.DMA((R - 1, NU)),
            pltpu.SemaphoreType.DMA((R - 1, NU)),
            pltpu.SemaphoreType.DMA((R - 1, NU)),
            pltpu.SemaphoreType.DMA((R - 1, NU)),
        ],
        compiler_params=pltpu.CompilerParams(collective_id=0),
    )(A, B)
